# Initial kernel scaffold; baseline (speedup 1.0000x reference)
import jax
import jax.numpy as jnp
from jax.experimental import pallas as pl


def kernel(x, hg, pd, Wt1, bt1, Wt2, bt2, Wv2e0, bv2e0, We2v0, be2v0, Wtl0, btl0, Wv2e1, bv2e1, We2v1, be2v1, Wtl1, btl1):
    raise NotImplementedError("write your pallas kernel here")


# speedup vs baseline: 2.2408x; 2.2408x over previous
import functools

import jax
import jax.numpy as jnp
from jax import lax
from jax.experimental import pallas as pl
from jax.experimental.pallas import tpu as pltpu
from jax.experimental.pallas import tpu_sc as plsc

NV = 10000
NE = 5000
NNZ = 320000
D = 128
NC = 2
NS = 16
NW = NC * NS
CH = 128
CPW = 80
NCHUNK = NW * CPW
NNZ_PAD = NCHUNK * CH
V_PAD = 10240
E_PAD = 5120
BM = 512

_F32 = jnp.float32


def _relu(v):
    return jnp.maximum(v, 0.0)


def _mesh():
    return plsc.VectorSubcoreMesh(core_axis_name="c", subcore_axis_name="s",
                                  num_cores=NC, num_subcores=NS)


def _zero_rows(rows_v):
    @pl.loop(0, 64)
    def _(i):
        for k in range(D // 16):
            rows_v[i, pl.ds(16 * k, 16)] = jnp.zeros((16,), _F32)


@functools.cache
def _sc_pass(dst_pad):
    rpt = dst_pad // NS

    def body(src, gidx, sidx, acc_out, gi_v, si_v, rows_v, acc_sh, sem):
        cid = lax.axis_index("c")
        sid = lax.axis_index("s")
        wid = cid * NS + sid

        _zero_rows(rows_v)

        @pl.loop(0, rpt // 64)
        def _(j):
            pltpu.sync_copy(rows_v.at[pl.ds(0, 64)],
                            acc_sh.at[pl.ds(sid * rpt + j * 64, 64)])

        plsc.subcore_barrier()

        @pl.loop(0, CPW)
        def _(j):
            ch = wid * CPW + j
            pltpu.sync_copy(gidx.at[ch], gi_v)
            pltpu.sync_copy(sidx.at[ch], si_v)
            pltpu.async_copy(src.at[gi_v], rows_v, sem).wait()
            pltpu.sync_copy(rows_v, acc_sh.at[si_v], add=True)

        plsc.subcore_barrier()

        @pl.loop(0, rpt // 64)
        def _(j):
            r0 = sid * rpt + j * 64
            pltpu.sync_copy(acc_sh.at[pl.ds(r0, 64)], rows_v.at[pl.ds(0, 64)])
            pltpu.sync_copy(rows_v.at[pl.ds(0, 64)],
                            acc_out.at[cid, pl.ds(r0, 64)])

    return pl.kernel(
        body,
        out_type=jax.ShapeDtypeStruct((NC, dst_pad, D), _F32),
        mesh=_mesh(),
        scratch_types=[
            pltpu.VMEM((CH,), jnp.int32),
            pltpu.VMEM((CH,), jnp.int32),
            pltpu.VMEM((CH, D), _F32),
            pltpu.VMEM_SHARED((dst_pad, D), _F32),
            pltpu.SemaphoreType.DMA,
        ],
    )


@functools.cache
def _sc_counts():
    rpt = V_PAD // NS
    cpt = NCHUNK // NS

    def body(eidx, vidx, cnt_out, si_v, rows_v, ones_v, cnt_sh):
        cid = lax.axis_index("c")
        sid = lax.axis_index("s")

        _zero_rows(rows_v)

        @pl.loop(0, CH)
        def _(i):
            for k in range(D // 16):
                ones_v[i, pl.ds(16 * k, 16)] = jnp.ones((16,), _F32)

        @pl.loop(0, rpt // 64)
        def _(j):
            pltpu.sync_copy(rows_v.at[pl.ds(0, 64)],
                            cnt_sh.at[pl.ds(sid * rpt + j * 64, 64)])

        plsc.subcore_barrier()

        @pl.loop(0, cpt)
        def _(j):
            ch = sid * cpt + j

            @pl.when(cid == 0)
            def _():
                pltpu.sync_copy(eidx.at[ch], si_v)

            @pl.when(cid == 1)
            def _():
                pltpu.sync_copy(vidx.at[ch], si_v)

            pltpu.sync_copy(ones_v, cnt_sh.at[si_v], add=True)

        plsc.subcore_barrier()

        @pl.loop(0, rpt // 64)
        def _(j):
            r0 = sid * rpt + j * 64
            pltpu.sync_copy(cnt_sh.at[pl.ds(r0, 64)], rows_v.at[pl.ds(0, 64)])
            pltpu.sync_copy(rows_v.at[pl.ds(0, 64)],
                            cnt_out.at[cid, pl.ds(r0, 64)])

    return pl.kernel(
        body,
        out_type=jax.ShapeDtypeStruct((NC, V_PAD, D), _F32),
        mesh=_mesh(),
        scratch_types=[
            pltpu.VMEM((CH,), jnp.int32),
            pltpu.VMEM((CH, D), _F32),
            pltpu.VMEM((CH, D), _F32),
            pltpu.VMEM_SHARED((V_PAD, D), _F32),
        ],
    )


def _dot(a, b):
    return jnp.dot(a, b, preferred_element_type=_F32)


def _a_body(pd_r, x_r, Wt1_r, bt1_r, Wt2_r, bt2_r, Wv_r, bv_r, topo_r, h0_r):
    t = _relu(_dot(pd_r[...], Wt1_r[...]) + bt1_r[...])
    topo_r[...] = _relu(_dot(t, Wt2_r[...]) + bt2_r[...])
    h0_r[...] = _relu(_dot(x_r[...], Wv_r[...]) + bv_r[...])


def _inv_counts(cnt_r):
    return 1.0 / jnp.maximum(cnt_r[...][:, 0], 1.0)


def _e_body(acc_r, ce_r, y_r):
    s = acc_r[0] + acc_r[1]
    y_r[...] = s * _inv_counts(ce_r)[:, None]


def _b_body(acc_r, cv_r, t_r, We_r, be_r, Wtl_r, btl_r, Wv_r, bv_r,
            t1_r, h1_r):
    z = (acc_r[0] + acc_r[1]) * _inv_counts(cv_r)[:, None]
    xc = _relu(_dot(z, We_r[...]) + be_r[...])
    t1 = _dot(t_r[...], Wtl_r[...]) + btl_r[...]
    t1_r[...] = t1
    xn = _relu(xc + xc * t1)
    h1_r[...] = _relu(_dot(xn, Wv_r[...]) + bv_r[...])


def _c_body(acc_r, cv_r, t_r, We_r, be_r, Wtl_r, btl_r, x2_r, t2_r):
    z = (acc_r[0] + acc_r[1]) * _inv_counts(cv_r)[:, None]
    xc = _relu(_dot(z, We_r[...]) + be_r[...])
    t2 = _dot(t_r[...], Wtl_r[...]) + btl_r[...]
    t2_r[...] = t2
    x2_r[...] = _relu(xc + xc * t2)


def _row_spec(w=D):
    return pl.BlockSpec((BM, w), lambda i: (i, 0))


def _full2(shape):
    return pl.BlockSpec(shape, lambda i: (0, 0))


def _bias_spec():
    return pl.BlockSpec((D,), lambda i: (0,))


def _part_spec(w=D):
    return pl.BlockSpec((NC, BM, w), lambda i: (0, i, 0))


_a_call = pl.pallas_call(
    _a_body,
    grid=(V_PAD // BM,),
    in_specs=[
        pl.BlockSpec((BM, 5), lambda i: (i, 0)),
        _row_spec(),
        _full2((5, D)), _bias_spec(),
        _full2((D, D)), _bias_spec(),
        _full2((D, D)), _bias_spec(),
    ],
    out_specs=[_row_spec(), _row_spec()],
    out_shape=[
        jax.ShapeDtypeStruct((V_PAD, D), _F32),
        jax.ShapeDtypeStruct((V_PAD, D), _F32),
    ],
)

_e_call = pl.pallas_call(
    _e_body,
    grid=(E_PAD // BM,),
    in_specs=[_part_spec(), _row_spec()],
    out_specs=_row_spec(),
    out_shape=jax.ShapeDtypeStruct((E_PAD, D), _F32),
)

_b_call = pl.pallas_call(
    _b_body,
    grid=(V_PAD // BM,),
    in_specs=[
        _part_spec(), _row_spec(), _row_spec(),
        _full2((D, D)), _bias_spec(),
        _full2((D, D)), _bias_spec(),
        _full2((D, D)), _bias_spec(),
    ],
    out_specs=[_row_spec(), _row_spec()],
    out_shape=[
        jax.ShapeDtypeStruct((V_PAD, D), _F32),
        jax.ShapeDtypeStruct((V_PAD, D), _F32),
    ],
)

_c_call = pl.pallas_call(
    _c_body,
    grid=(V_PAD // BM,),
    in_specs=[
        _part_spec(), _row_spec(), _row_spec(),
        _full2((D, D)), _bias_spec(),
        _full2((D, D)), _bias_spec(),
    ],
    out_specs=[_row_spec(), _row_spec()],
    out_shape=[
        jax.ShapeDtypeStruct((V_PAD, D), _F32),
        jax.ShapeDtypeStruct((V_PAD, D), _F32),
    ],
)


def kernel(x, hg, pd, Wt1, bt1, Wt2, bt2, Wv2e0, bv2e0, We2v0, be2v0,
           Wtl0, btl0, Wv2e1, bv2e1, We2v1, be2v1, Wtl1, btl1):
    padn = NNZ_PAD - NNZ
    vp = jnp.concatenate(
        [hg[0], jnp.full((padn,), NV, jnp.int32)]).reshape(NCHUNK, CH)
    ep = jnp.concatenate(
        [hg[1], jnp.full((padn,), NE, jnp.int32)]).reshape(NCHUNK, CH)

    topo1, h0 = _a_call(pd, x, Wt1, bt1, Wt2, bt2, Wv2e0, bv2e0)
    cnt = _sc_counts()(ep, vp)
    ce, cv = cnt[0], cnt[1]

    acc_e0 = _sc_pass(E_PAD)(h0, vp, ep)
    y0 = _e_call(acc_e0, ce[:E_PAD])
    acc_v0 = _sc_pass(V_PAD)(y0, ep, vp)
    t1, h1 = _b_call(acc_v0, cv, topo1,
                     We2v0, be2v0, Wtl0, btl0, Wv2e1, bv2e1)

    acc_e1 = _sc_pass(E_PAD)(h1, vp, ep)
    y1 = _e_call(acc_e1, ce[:E_PAD])
    acc_v1 = _sc_pass(V_PAD)(y1, ep, vp)
    x2, t2 = _c_call(acc_v1, cv,
                     t1, We2v1, be2v1, Wtl1, btl1)

    return (x2[:NV], t2[:NV])


# speedup vs baseline: 2.8223x; 1.2595x over previous
import functools

import jax
import jax.numpy as jnp
from jax import lax
from jax.experimental import pallas as pl
from jax.experimental.pallas import tpu as pltpu
from jax.experimental.pallas import tpu_sc as plsc

NV = 10000
NE = 5000
NNZ = 320000
D = 128
NC = 2
NS = 16
NW = NC * NS
CH = 128
CPW = 80
NCHUNK = NW * CPW
NNZ_PAD = NCHUNK * CH
V_PAD = 10240
E_PAD = 5120
BM = 512

_F32 = jnp.float32


def _relu(v):
    return jnp.maximum(v, 0.0)


def _mesh():
    return plsc.VectorSubcoreMesh(core_axis_name="c", subcore_axis_name="s",
                                  num_cores=NC, num_subcores=NS)


def _zero_rows(rows_v):
    @pl.loop(0, 64)
    def _(i):
        for k in range(D // 16):
            rows_v[i, pl.ds(16 * k, 16)] = jnp.zeros((16,), _F32)


HALF = CPW // 2


@functools.cache
def _sc_pass(dst_pad):
    rpt = dst_pad // NS

    def body(src, gidx, sidx, acc_out,
             gi_all, si_all, rows0, rows1, acc_sh, sem0, sem1):
        cid = lax.axis_index("c")
        sid = lax.axis_index("s")
        wid = cid * NS + sid

        _zero_rows(rows0)

        @pl.loop(0, rpt // 64)
        def _(j):
            pltpu.sync_copy(rows0.at[pl.ds(0, 64)],
                            acc_sh.at[pl.ds(sid * rpt + j * 64, 64)])

        plsc.subcore_barrier()

        for half in range(CPW // HALF):
            base = wid * CPW + half * HALF
            pltpu.sync_copy(gidx.at[pl.ds(base, HALF)], gi_all)
            pltpu.sync_copy(sidx.at[pl.ds(base, HALF)], si_all)
            pltpu.async_copy(src.at[gi_all.at[0]], rows0, sem0)

            @pl.loop(0, HALF // 2)
            def _(jj):
                j0 = 2 * jj
                pltpu.async_copy(src.at[gi_all.at[j0 + 1]], rows1, sem1)
                pltpu.make_async_copy(src.at[gi_all.at[j0]], rows0,
                                      sem0).wait()
                pltpu.sync_copy(rows0, acc_sh.at[si_all.at[j0]], add=True)

                @pl.when(jj < HALF // 2 - 1)
                def _():
                    pltpu.async_copy(src.at[gi_all.at[j0 + 2]], rows0, sem0)

                pltpu.make_async_copy(src.at[gi_all.at[j0 + 1]], rows1,
                                      sem1).wait()
                pltpu.sync_copy(rows1, acc_sh.at[si_all.at[j0 + 1]], add=True)

        plsc.subcore_barrier()

        @pl.loop(0, rpt // 64)
        def _(j):
            r0 = sid * rpt + j * 64
            pltpu.sync_copy(acc_sh.at[pl.ds(r0, 64)], rows0.at[pl.ds(0, 64)])
            pltpu.sync_copy(rows0.at[pl.ds(0, 64)],
                            acc_out.at[cid, pl.ds(r0, 64)])

    return pl.kernel(
        body,
        out_type=jax.ShapeDtypeStruct((NC, dst_pad, D), _F32),
        mesh=_mesh(),
        scratch_types=[
            pltpu.VMEM((HALF, CH), jnp.int32),
            pltpu.VMEM((HALF, CH), jnp.int32),
            pltpu.VMEM((CH, D), _F32),
            pltpu.VMEM((CH, D), _F32),
            pltpu.VMEM_SHARED((dst_pad, D), _F32),
            pltpu.SemaphoreType.DMA,
            pltpu.SemaphoreType.DMA,
        ],
    )


@functools.cache
def _sc_counts():
    rpt = V_PAD // NS
    cpt = NCHUNK // NS

    def body(eidx, vidx, cnt_out, si_v, rows_v, ones_v, cnt_sh):
        cid = lax.axis_index("c")
        sid = lax.axis_index("s")

        _zero_rows(rows_v)

        @pl.loop(0, CH)
        def _(i):
            for k in range(D // 16):
                ones_v[i, pl.ds(16 * k, 16)] = jnp.ones((16,), _F32)

        @pl.loop(0, rpt // 64)
        def _(j):
            pltpu.sync_copy(rows_v.at[pl.ds(0, 64)],
                            cnt_sh.at[pl.ds(sid * rpt + j * 64, 64)])

        plsc.subcore_barrier()

        @pl.loop(0, cpt)
        def _(j):
            ch = sid * cpt + j

            @pl.when(cid == 0)
            def _():
                pltpu.sync_copy(eidx.at[ch], si_v)

            @pl.when(cid == 1)
            def _():
                pltpu.sync_copy(vidx.at[ch], si_v)

            pltpu.sync_copy(ones_v, cnt_sh.at[si_v], add=True)

        plsc.subcore_barrier()

        @pl.loop(0, rpt // 64)
        def _(j):
            r0 = sid * rpt + j * 64
            pltpu.sync_copy(cnt_sh.at[pl.ds(r0, 64)], rows_v.at[pl.ds(0, 64)])
            pltpu.sync_copy(rows_v.at[pl.ds(0, 64)],
                            cnt_out.at[cid, pl.ds(r0, 64)])

    return pl.kernel(
        body,
        out_type=jax.ShapeDtypeStruct((NC, V_PAD, D), _F32),
        mesh=_mesh(),
        scratch_types=[
            pltpu.VMEM((CH,), jnp.int32),
            pltpu.VMEM((CH, D), _F32),
            pltpu.VMEM((CH, D), _F32),
            pltpu.VMEM_SHARED((V_PAD, D), _F32),
        ],
    )


def _dot(a, b):
    return jnp.dot(a, b, preferred_element_type=_F32)


def _a_body(pd_r, x_r, Wt1_r, bt1_r, Wt2_r, bt2_r, Wv_r, bv_r, topo_r, h0_r):
    t = _relu(_dot(pd_r[...], Wt1_r[...]) + bt1_r[...])
    topo_r[...] = _relu(_dot(t, Wt2_r[...]) + bt2_r[...])
    h0_r[...] = _relu(_dot(x_r[...], Wv_r[...]) + bv_r[...])


def _inv_counts(cnt_r):
    return 1.0 / jnp.maximum(cnt_r[...][:, 0], 1.0)


def _e_body(acc_r, ce_r, y_r):
    s = acc_r[0] + acc_r[1]
    y_r[...] = s * _inv_counts(ce_r)[:, None]


def _b_body(acc_r, cv_r, t_r, We_r, be_r, Wtl_r, btl_r, Wv_r, bv_r,
            t1_r, h1_r):
    z = (acc_r[0] + acc_r[1]) * _inv_counts(cv_r)[:, None]
    xc = _relu(_dot(z, We_r[...]) + be_r[...])
    t1 = _dot(t_r[...], Wtl_r[...]) + btl_r[...]
    t1_r[...] = t1
    xn = _relu(xc + xc * t1)
    h1_r[...] = _relu(_dot(xn, Wv_r[...]) + bv_r[...])


def _c_body(acc_r, cv_r, t_r, We_r, be_r, Wtl_r, btl_r, x2_r, t2_r):
    z = (acc_r[0] + acc_r[1]) * _inv_counts(cv_r)[:, None]
    xc = _relu(_dot(z, We_r[...]) + be_r[...])
    t2 = _dot(t_r[...], Wtl_r[...]) + btl_r[...]
    t2_r[...] = t2
    x2_r[...] = _relu(xc + xc * t2)


def _row_spec(w=D):
    return pl.BlockSpec((BM, w), lambda i: (i, 0))


def _full2(shape):
    return pl.BlockSpec(shape, lambda i: (0, 0))


def _bias_spec():
    return pl.BlockSpec((D,), lambda i: (0,))


def _part_spec(w=D):
    return pl.BlockSpec((NC, BM, w), lambda i: (0, i, 0))


_a_call = pl.pallas_call(
    _a_body,
    grid=(V_PAD // BM,),
    in_specs=[
        pl.BlockSpec((BM, 5), lambda i: (i, 0)),
        _row_spec(),
        _full2((5, D)), _bias_spec(),
        _full2((D, D)), _bias_spec(),
        _full2((D, D)), _bias_spec(),
    ],
    out_specs=[_row_spec(), _row_spec()],
    out_shape=[
        jax.ShapeDtypeStruct((V_PAD, D), _F32),
        jax.ShapeDtypeStruct((V_PAD, D), _F32),
    ],
)

_e_call = pl.pallas_call(
    _e_body,
    grid=(E_PAD // BM,),
    in_specs=[_part_spec(), _row_spec()],
    out_specs=_row_spec(),
    out_shape=jax.ShapeDtypeStruct((E_PAD, D), _F32),
)

_b_call = pl.pallas_call(
    _b_body,
    grid=(V_PAD // BM,),
    in_specs=[
        _part_spec(), _row_spec(), _row_spec(),
        _full2((D, D)), _bias_spec(),
        _full2((D, D)), _bias_spec(),
        _full2((D, D)), _bias_spec(),
    ],
    out_specs=[_row_spec(), _row_spec()],
    out_shape=[
        jax.ShapeDtypeStruct((V_PAD, D), _F32),
        jax.ShapeDtypeStruct((V_PAD, D), _F32),
    ],
)

_c_call = pl.pallas_call(
    _c_body,
    grid=(V_PAD // BM,),
    in_specs=[
        _part_spec(), _row_spec(), _row_spec(),
        _full2((D, D)), _bias_spec(),
        _full2((D, D)), _bias_spec(),
    ],
    out_specs=[_row_spec(), _row_spec()],
    out_shape=[
        jax.ShapeDtypeStruct((V_PAD, D), _F32),
        jax.ShapeDtypeStruct((V_PAD, D), _F32),
    ],
)


def kernel(x, hg, pd, Wt1, bt1, Wt2, bt2, Wv2e0, bv2e0, We2v0, be2v0,
           Wtl0, btl0, Wv2e1, bv2e1, We2v1, be2v1, Wtl1, btl1):
    padn = NNZ_PAD - NNZ
    vp = jnp.concatenate(
        [hg[0], jnp.full((padn,), NV, jnp.int32)]).reshape(NCHUNK, CH)
    ep = jnp.concatenate(
        [hg[1], jnp.full((padn,), NE, jnp.int32)]).reshape(NCHUNK, CH)

    topo1, h0 = _a_call(pd, x, Wt1, bt1, Wt2, bt2, Wv2e0, bv2e0)
    cnt = _sc_counts()(ep, vp)
    ce, cv = cnt[0], cnt[1]

    acc_e0 = _sc_pass(E_PAD)(h0, vp, ep)
    y0 = _e_call(acc_e0, ce[:E_PAD])
    acc_v0 = _sc_pass(V_PAD)(y0, ep, vp)
    t1, h1 = _b_call(acc_v0, cv, topo1,
                     We2v0, be2v0, Wtl0, btl0, Wv2e1, bv2e1)

    acc_e1 = _sc_pass(E_PAD)(h1, vp, ep)
    y1 = _e_call(acc_e1, ce[:E_PAD])
    acc_v1 = _sc_pass(V_PAD)(y1, ep, vp)
    x2, t2 = _c_call(acc_v1, cv,
                     t1, We2v1, be2v1, Wtl1, btl1)

    return (x2[:NV], t2[:NV])


# speedup vs baseline: 8.4579x; 2.9968x over previous
import functools

import jax
import jax.numpy as jnp
from jax import lax
from jax.experimental import pallas as pl
from jax.experimental.pallas import tpu as pltpu
from jax.experimental.pallas import tpu_sc as plsc

NV = 10000
NE = 5000
NNZ = 320000
D = 128
NC = 2
NS = 16
NW = NC * NS
CH = 128
CPW = 80
NCHUNK = NW * CPW
NNZ_PAD = NCHUNK * CH
V_PAD = 10240
E_PAD = 5120
BM = 512

_F32 = jnp.float32


def _relu(v):
    return jnp.maximum(v, 0.0)


def _mesh():
    return plsc.VectorSubcoreMesh(core_axis_name="c", subcore_axis_name="s",
                                  num_cores=NC, num_subcores=NS)


def _zero_rows(rows_v):
    @pl.loop(0, 64)
    def _(i):
        for k in range(D // 16):
            rows_v[i, pl.ds(16 * k, 16)] = jnp.zeros((16,), _F32)


HALF = CPW // 2


@functools.cache
def _sc_pass(dst_pad):
    rpt = dst_pad // NS

    def body(src, gidx, sidx, acc_out,
             gi_all, si_all, rows0, rows1, acc_sh, sem0, sem1):
        cid = lax.axis_index("c")
        sid = lax.axis_index("s")
        wid = cid * NS + sid

        _zero_rows(rows0)

        @pl.loop(0, rpt // 64)
        def _(j):
            pltpu.sync_copy(rows0.at[pl.ds(0, 64)],
                            acc_sh.at[pl.ds(sid * rpt + j * 64, 64)])

        plsc.subcore_barrier()

        for half in range(CPW // HALF):
            base = wid * CPW + half * HALF
            pltpu.sync_copy(gidx.at[pl.ds(base, HALF)], gi_all)
            pltpu.sync_copy(sidx.at[pl.ds(base, HALF)], si_all)
            pltpu.async_copy(src.at[gi_all.at[0]], rows0, sem0)

            @pl.loop(0, HALF // 2)
            def _(jj):
                j0 = 2 * jj
                pltpu.async_copy(src.at[gi_all.at[j0 + 1]], rows1, sem1)
                pltpu.make_async_copy(src.at[gi_all.at[j0]], rows0,
                                      sem0).wait()
                pltpu.sync_copy(rows0, acc_sh.at[si_all.at[j0]], add=True)

                @pl.when(jj < HALF // 2 - 1)
                def _():
                    pltpu.async_copy(src.at[gi_all.at[j0 + 2]], rows0, sem0)

                pltpu.make_async_copy(src.at[gi_all.at[j0 + 1]], rows1,
                                      sem1).wait()
                pltpu.sync_copy(rows1, acc_sh.at[si_all.at[j0 + 1]], add=True)

        plsc.subcore_barrier()

        @pl.loop(0, rpt // 64)
        def _(j):
            r0 = sid * rpt + j * 64
            pltpu.sync_copy(acc_sh.at[pl.ds(r0, 64)], rows0.at[pl.ds(0, 64)])
            pltpu.sync_copy(rows0.at[pl.ds(0, 64)],
                            acc_out.at[cid, pl.ds(r0, 64)])

    return pl.kernel(
        body,
        out_type=jax.ShapeDtypeStruct((NC, dst_pad, D), _F32),
        mesh=_mesh(),
        scratch_types=[
            pltpu.VMEM((HALF, CH), jnp.int32),
            pltpu.VMEM((HALF, CH), jnp.int32),
            pltpu.VMEM((CH, D), _F32),
            pltpu.VMEM((CH, D), _F32),
            pltpu.VMEM_SHARED((dst_pad, D), _F32),
            pltpu.SemaphoreType.DMA,
            pltpu.SemaphoreType.DMA,
        ],
    )


@functools.cache
def _sc_counts():
    rpt = V_PAD // NS
    cpt = NCHUNK // NS

    def body(eidx, vidx, cnt_out, si_v, rows_v, ones_v, cnt_sh):
        cid = lax.axis_index("c")
        sid = lax.axis_index("s")

        _zero_rows(rows_v)

        @pl.loop(0, CH)
        def _(i):
            for k in range(D // 16):
                ones_v[i, pl.ds(16 * k, 16)] = jnp.ones((16,), _F32)

        @pl.loop(0, rpt // 64)
        def _(j):
            pltpu.sync_copy(rows_v.at[pl.ds(0, 64)],
                            cnt_sh.at[pl.ds(sid * rpt + j * 64, 64)])

        plsc.subcore_barrier()

        @pl.loop(0, cpt)
        def _(j):
            ch = sid * cpt + j

            @pl.when(cid == 0)
            def _():
                pltpu.sync_copy(eidx.at[ch], si_v)

            @pl.when(cid == 1)
            def _():
                pltpu.sync_copy(vidx.at[ch], si_v)

            pltpu.sync_copy(ones_v, cnt_sh.at[si_v], add=True)

        plsc.subcore_barrier()

        @pl.loop(0, rpt // 64)
        def _(j):
            r0 = sid * rpt + j * 64
            pltpu.sync_copy(cnt_sh.at[pl.ds(r0, 64)], rows_v.at[pl.ds(0, 64)])
            pltpu.sync_copy(rows_v.at[pl.ds(0, 64)],
                            cnt_out.at[cid, pl.ds(r0, 64)])

    return pl.kernel(
        body,
        out_type=jax.ShapeDtypeStruct((NC, V_PAD, D), _F32),
        mesh=_mesh(),
        scratch_types=[
            pltpu.VMEM((CH,), jnp.int32),
            pltpu.VMEM((CH, D), _F32),
            pltpu.VMEM((CH, D), _F32),
            pltpu.VMEM_SHARED((V_PAD, D), _F32),
        ],
    )


def _dot(a, b):
    return jnp.dot(a, b, preferred_element_type=_F32)


def _a_body(pd_r, x_r, Wt1_r, bt1_r, Wt2_r, bt2_r, Wv_r, bv_r, topo_r, h0_r):
    t = _relu(_dot(pd_r[...], Wt1_r[...]) + bt1_r[...])
    topo_r[...] = _relu(_dot(t, Wt2_r[...]) + bt2_r[...])
    h0_r[...] = _relu(_dot(x_r[...], Wv_r[...]) + bv_r[...])


def _inv_counts(cnt_r):
    return 1.0 / jnp.maximum(cnt_r[...][:, 0], 1.0)


def _e_body(acc_r, ce_r, y_r):
    s = acc_r[0] + acc_r[1]
    y_r[...] = s * _inv_counts(ce_r)[:, None]


def _b_body(acc_r, cv_r, t_r, We_r, be_r, Wtl_r, btl_r, Wv_r, bv_r,
            t1_r, h1_r):
    z = (acc_r[0] + acc_r[1]) * _inv_counts(cv_r)[:, None]
    xc = _relu(_dot(z, We_r[...]) + be_r[...])
    t1 = _dot(t_r[...], Wtl_r[...]) + btl_r[...]
    t1_r[...] = t1
    xn = _relu(xc + xc * t1)
    h1_r[...] = _relu(_dot(xn, Wv_r[...]) + bv_r[...])


def _c_body(acc_r, cv_r, t_r, We_r, be_r, Wtl_r, btl_r, x2_r, t2_r):
    z = (acc_r[0] + acc_r[1]) * _inv_counts(cv_r)[:, None]
    xc = _relu(_dot(z, We_r[...]) + be_r[...])
    t2 = _dot(t_r[...], Wtl_r[...]) + btl_r[...]
    t2_r[...] = t2
    x2_r[...] = _relu(xc + xc * t2)


def _row_spec(w=D):
    return pl.BlockSpec((BM, w), lambda i: (i, 0))


def _full2(shape):
    return pl.BlockSpec(shape, lambda i: (0, 0))


def _bias_spec():
    return pl.BlockSpec((D,), lambda i: (0,))


def _part_spec(w=D):
    return pl.BlockSpec((NC, BM, w), lambda i: (0, i, 0))


_a_call = pl.pallas_call(
    _a_body,
    grid=(V_PAD // BM,),
    in_specs=[
        pl.BlockSpec((BM, 5), lambda i: (i, 0)),
        _row_spec(),
        _full2((5, D)), _bias_spec(),
        _full2((D, D)), _bias_spec(),
        _full2((D, D)), _bias_spec(),
    ],
    out_specs=[_row_spec(), _row_spec()],
    out_shape=[
        jax.ShapeDtypeStruct((V_PAD, D), _F32),
        jax.ShapeDtypeStruct((V_PAD, D), _F32),
    ],
)

_e_call = pl.pallas_call(
    _e_body,
    grid=(E_PAD // BM,),
    in_specs=[_part_spec(), _row_spec()],
    out_specs=_row_spec(),
    out_shape=jax.ShapeDtypeStruct((E_PAD, D), _F32),
)

_b_call = pl.pallas_call(
    _b_body,
    grid=(V_PAD // BM,),
    in_specs=[
        _part_spec(), _row_spec(), _row_spec(),
        _full2((D, D)), _bias_spec(),
        _full2((D, D)), _bias_spec(),
        _full2((D, D)), _bias_spec(),
    ],
    out_specs=[_row_spec(), _row_spec()],
    out_shape=[
        jax.ShapeDtypeStruct((V_PAD, D), _F32),
        jax.ShapeDtypeStruct((V_PAD, D), _F32),
    ],
)

_c_call = pl.pallas_call(
    _c_body,
    grid=(V_PAD // BM,),
    in_specs=[
        _part_spec(), _row_spec(), _row_spec(),
        _full2((D, D)), _bias_spec(),
        _full2((D, D)), _bias_spec(),
    ],
    out_specs=[_row_spec(), _row_spec()],
    out_shape=[
        jax.ShapeDtypeStruct((V_PAD, D), _F32),
        jax.ShapeDtypeStruct((V_PAD, D), _F32),
    ],
)


def kernel(x, hg, pd, Wt1, bt1, Wt2, bt2, Wv2e0, bv2e0, We2v0, be2v0,
           Wtl0, btl0, Wv2e1, bv2e1, We2v1, be2v1, Wtl1, btl1):
    padn = NNZ_PAD - NNZ
    arp = jnp.arange(padn, dtype=jnp.int32)
    vp = jnp.concatenate(
        [hg[0], NV + arp % (V_PAD - NV)]).reshape(NCHUNK, CH)
    ep = jnp.concatenate(
        [hg[1], NE + arp % (E_PAD - NE)]).reshape(NCHUNK, CH)

    topo1, h0 = _a_call(pd, x, Wt1, bt1, Wt2, bt2, Wv2e0, bv2e0)
    cnt = _sc_counts()(ep, vp)
    ce, cv = cnt[0], cnt[1]

    acc_e0 = _sc_pass(E_PAD)(h0, vp, ep)
    y0 = _e_call(acc_e0, ce[:E_PAD])
    acc_v0 = _sc_pass(V_PAD)(y0, ep, vp)
    t1, h1 = _b_call(acc_v0, cv, topo1,
                     We2v0, be2v0, Wtl0, btl0, Wv2e1, bv2e1)

    acc_e1 = _sc_pass(E_PAD)(h1, vp, ep)
    y1 = _e_call(acc_e1, ce[:E_PAD])
    acc_v1 = _sc_pass(V_PAD)(y1, ep, vp)
    x2, t2 = _c_call(acc_v1, cv,
                     t1, We2v1, be2v1, Wtl1, btl1)

    return (x2[:NV], t2[:NV])


# speedup vs baseline: 9.2944x; 1.0989x over previous
import functools

import jax
import jax.numpy as jnp
from jax import lax
from jax.experimental import pallas as pl
from jax.experimental.pallas import tpu as pltpu
from jax.experimental.pallas import tpu_sc as plsc

NV = 10000
NE = 5000
NNZ = 320000
D = 128
NC = 2
NS = 16
NW = NC * NS
CH = 128
CPW = 80
NCHUNK = NW * CPW
NNZ_PAD = NCHUNK * CH
V_PAD = 10240
E_PAD = 5120
BM = 512

_F32 = jnp.float32


def _relu(v):
    return jnp.maximum(v, 0.0)


def _mesh():
    return plsc.VectorSubcoreMesh(core_axis_name="c", subcore_axis_name="s",
                                  num_cores=NC, num_subcores=NS)


def _zero_rows(rows_v):
    @pl.loop(0, 64)
    def _(i):
        for k in range(D // 16):
            rows_v[i, pl.ds(16 * k, 16)] = jnp.zeros((16,), _F32)


HALF = CPW // 2


@functools.cache
def _sc_pass(dst_pad):
    rpt = dst_pad // NS

    def body(src, gidx, sidx, acc_out,
             gi_all, si_all, rows0, rows1, acc_sh, sem0, sem1):
        cid = lax.axis_index("c")
        sid = lax.axis_index("s")
        wid = cid * NS + sid

        _zero_rows(rows0)

        @pl.loop(0, rpt // 64)
        def _(j):
            pltpu.sync_copy(rows0.at[pl.ds(0, 64)],
                            acc_sh.at[pl.ds(sid * rpt + j * 64, 64)])

        plsc.subcore_barrier()

        for half in range(CPW // HALF):
            base = wid * CPW + half * HALF
            pltpu.sync_copy(gidx.at[pl.ds(base, HALF)], gi_all)
            pltpu.sync_copy(sidx.at[pl.ds(base, HALF)], si_all)
            pltpu.async_copy(src.at[gi_all.at[0]], rows0, sem0)

            @pl.loop(0, HALF // 2)
            def _(jj):
                j0 = 2 * jj
                pltpu.async_copy(src.at[gi_all.at[j0 + 1]], rows1, sem1)
                pltpu.make_async_copy(src.at[gi_all.at[j0]], rows0,
                                      sem0).wait()
                pltpu.sync_copy(rows0, acc_sh.at[si_all.at[j0]], add=True)

                @pl.when(jj < HALF // 2 - 1)
                def _():
                    pltpu.async_copy(src.at[gi_all.at[j0 + 2]], rows0, sem0)

                pltpu.make_async_copy(src.at[gi_all.at[j0 + 1]], rows1,
                                      sem1).wait()
                pltpu.sync_copy(rows1, acc_sh.at[si_all.at[j0 + 1]], add=True)

        plsc.subcore_barrier()

        @pl.loop(0, rpt // 64)
        def _(j):
            r0 = sid * rpt + j * 64
            pltpu.sync_copy(acc_sh.at[pl.ds(r0, 64)], rows0.at[pl.ds(0, 64)])
            pltpu.sync_copy(rows0.at[pl.ds(0, 64)],
                            acc_out.at[cid, pl.ds(r0, 64)])

    return pl.kernel(
        body,
        out_type=jax.ShapeDtypeStruct((NC, dst_pad, D), _F32),
        mesh=_mesh(),
        scratch_types=[
            pltpu.VMEM((HALF, CH), jnp.int32),
            pltpu.VMEM((HALF, CH), jnp.int32),
            pltpu.VMEM((CH, D), _F32),
            pltpu.VMEM((CH, D), _F32),
            pltpu.VMEM_SHARED((dst_pad, D), _F32),
            pltpu.SemaphoreType.DMA,
            pltpu.SemaphoreType.DMA,
        ],
    )


CW = 128


@functools.cache
def _sc_counts():
    rpt = V_PAD // NS
    cpt = NCHUNK // NS

    def body(idx2, cnt_out, si_all, z_v, ones_v, cnt_sh):
        cid = lax.axis_index("c")
        sid = lax.axis_index("s")

        @pl.loop(0, CH)
        def _(i):
            for k in range(CW // 16):
                z_v[i, pl.ds(16 * k, 16)] = jnp.zeros((16,), _F32)
                ones_v[i, pl.ds(16 * k, 16)] = jnp.ones((16,), _F32)

        @pl.loop(0, rpt // CH)
        def _(j):
            pltpu.sync_copy(z_v, cnt_sh.at[pl.ds(sid * rpt + j * CH, CH)])

        plsc.subcore_barrier()

        for b in range(cpt // HALF):
            base = sid * cpt + b * HALF
            pltpu.sync_copy(idx2.at[cid, pl.ds(base, HALF)], si_all)

            @pl.loop(0, HALF)
            def _(j):
                pltpu.sync_copy(ones_v, cnt_sh.at[si_all.at[j]], add=True)

        plsc.subcore_barrier()

        @pl.loop(0, rpt // CH)
        def _(j):
            r0 = sid * rpt + j * CH
            pltpu.sync_copy(cnt_sh.at[pl.ds(r0, CH)], z_v)
            pltpu.sync_copy(z_v, cnt_out.at[cid, pl.ds(r0, CH)])

    return pl.kernel(
        body,
        out_type=jax.ShapeDtypeStruct((NC, V_PAD, CW), _F32),
        mesh=_mesh(),
        scratch_types=[
            pltpu.VMEM((HALF, CH), jnp.int32),
            pltpu.VMEM((CH, CW), _F32),
            pltpu.VMEM((CH, CW), _F32),
            pltpu.VMEM_SHARED((V_PAD, CW), _F32),
        ],
    )


def _dot(a, b):
    return jnp.dot(a, b, preferred_element_type=_F32)


def _a_body(pd_r, x_r, Wt1_r, bt1_r, Wt2_r, bt2_r, Wv_r, bv_r, topo_r, h0_r):
    t = _relu(_dot(pd_r[...], Wt1_r[...]) + bt1_r[...])
    topo_r[...] = _relu(_dot(t, Wt2_r[...]) + bt2_r[...])
    h0_r[...] = _relu(_dot(x_r[...], Wv_r[...]) + bv_r[...])


def _inv_counts(cnt_r):
    return 1.0 / jnp.maximum(cnt_r[...][:, 0], 1.0)


def _e_body(acc_r, ce_r, y_r):
    s = acc_r[0] + acc_r[1]
    y_r[...] = s * _inv_counts(ce_r)[:, None]


def _b_body(acc_r, cv_r, t_r, We_r, be_r, Wtl_r, btl_r, Wv_r, bv_r,
            t1_r, h1_r):
    z = (acc_r[0] + acc_r[1]) * _inv_counts(cv_r)[:, None]
    xc = _relu(_dot(z, We_r[...]) + be_r[...])
    t1 = _dot(t_r[...], Wtl_r[...]) + btl_r[...]
    t1_r[...] = t1
    xn = _relu(xc + xc * t1)
    h1_r[...] = _relu(_dot(xn, Wv_r[...]) + bv_r[...])


def _c_body(acc_r, cv_r, t_r, We_r, be_r, Wtl_r, btl_r, x2_r, t2_r):
    z = (acc_r[0] + acc_r[1]) * _inv_counts(cv_r)[:, None]
    xc = _relu(_dot(z, We_r[...]) + be_r[...])
    t2 = _dot(t_r[...], Wtl_r[...]) + btl_r[...]
    t2_r[...] = t2
    x2_r[...] = _relu(xc + xc * t2)


def _row_spec(w=D):
    return pl.BlockSpec((BM, w), lambda i: (i, 0))


def _full2(shape):
    return pl.BlockSpec(shape, lambda i: (0, 0))


def _bias_spec():
    return pl.BlockSpec((D,), lambda i: (0,))


def _part_spec(w=D):
    return pl.BlockSpec((NC, BM, w), lambda i: (0, i, 0))


_a_call = pl.pallas_call(
    _a_body,
    grid=(V_PAD // BM,),
    in_specs=[
        pl.BlockSpec((BM, 5), lambda i: (i, 0)),
        _row_spec(),
        _full2((5, D)), _bias_spec(),
        _full2((D, D)), _bias_spec(),
        _full2((D, D)), _bias_spec(),
    ],
    out_specs=[_row_spec(), _row_spec()],
    out_shape=[
        jax.ShapeDtypeStruct((V_PAD, D), _F32),
        jax.ShapeDtypeStruct((V_PAD, D), _F32),
    ],
)

_e_call = pl.pallas_call(
    _e_body,
    grid=(E_PAD // BM,),
    in_specs=[_part_spec(), _row_spec(CW)],
    out_specs=_row_spec(),
    out_shape=jax.ShapeDtypeStruct((E_PAD, D), _F32),
)

_b_call = pl.pallas_call(
    _b_body,
    grid=(V_PAD // BM,),
    in_specs=[
        _part_spec(), _row_spec(CW), _row_spec(),
        _full2((D, D)), _bias_spec(),
        _full2((D, D)), _bias_spec(),
        _full2((D, D)), _bias_spec(),
    ],
    out_specs=[_row_spec(), _row_spec()],
    out_shape=[
        jax.ShapeDtypeStruct((V_PAD, D), _F32),
        jax.ShapeDtypeStruct((V_PAD, D), _F32),
    ],
)

_c_call = pl.pallas_call(
    _c_body,
    grid=(V_PAD // BM,),
    in_specs=[
        _part_spec(), _row_spec(CW), _row_spec(),
        _full2((D, D)), _bias_spec(),
        _full2((D, D)), _bias_spec(),
    ],
    out_specs=[_row_spec(), _row_spec()],
    out_shape=[
        jax.ShapeDtypeStruct((V_PAD, D), _F32),
        jax.ShapeDtypeStruct((V_PAD, D), _F32),
    ],
)


def kernel(x, hg, pd, Wt1, bt1, Wt2, bt2, Wv2e0, bv2e0, We2v0, be2v0,
           Wtl0, btl0, Wv2e1, bv2e1, We2v1, be2v1, Wtl1, btl1):
    padn = NNZ_PAD - NNZ
    arp = jnp.arange(padn, dtype=jnp.int32)
    vp = jnp.concatenate(
        [hg[0], NV + arp % (V_PAD - NV)]).reshape(NCHUNK, CH)
    ep = jnp.concatenate(
        [hg[1], NE + arp % (E_PAD - NE)]).reshape(NCHUNK, CH)

    topo1, h0 = _a_call(pd, x, Wt1, bt1, Wt2, bt2, Wv2e0, bv2e0)
    cnt = _sc_counts()(jnp.stack([ep, vp]))
    ce, cv = cnt[0], cnt[1]

    acc_e0 = _sc_pass(E_PAD)(h0, vp, ep)
    y0 = _e_call(acc_e0, ce[:E_PAD])
    acc_v0 = _sc_pass(V_PAD)(y0, ep, vp)
    t1, h1 = _b_call(acc_v0, cv, topo1,
                     We2v0, be2v0, Wtl0, btl0, Wv2e1, bv2e1)

    acc_e1 = _sc_pass(E_PAD)(h1, vp, ep)
    y1 = _e_call(acc_e1, ce[:E_PAD])
    acc_v1 = _sc_pass(V_PAD)(y1, ep, vp)
    x2, t2 = _c_call(acc_v1, cv,
                     t1, We2v1, be2v1, Wtl1, btl1)

    return (x2[:NV], t2[:NV])
